# baseline (speedup 1.0000x reference)
import functools

import jax
import jax.numpy as jnp
from jax import lax
from jax.experimental import pallas as pl
from jax.experimental.pallas import tpu as pltpu
from jax.experimental.pallas import tpu_sc as plsc

N_NODES = 10000
N_EDGES = 320000
D = 128
EPS = 1e-5

NC = 2
NS = 16
NW = NC * NS
CHUNK = 80
N_CHUNKS = 128
E_PAD = NW * N_CHUNKS * CHUNK
N_PAD = 10240
ROWS_PER_TILE = N_PAD // NS
ZROWS = CHUNK
LANES = 16
GROUPS = CHUNK // LANES
NBUF = 3
MAIN_T = N_CHUNKS // NBUF


def _sc_body(z_hbm, rec_hbm, wt_hbm, out_hbm,
             rows0, rows1, rows2, rec0, rec1, rec2, w0, w1, w2, sidx, acc,
             g0, g1, g2, s0, s1, s2, i0, i1, i2):
    c = lax.axis_index("c")
    s = lax.axis_index("s")
    wid = s * NC + c
    base = wid * N_CHUNKS

    rows = (rows0, rows1, rows2)
    recb = (rec0, rec1, rec2)
    wbuf = (w0, w1, w2)
    gsem = (g0, g1, g2)
    ssem = (s0, s1, s2)
    isem = (i0, i1, i2)

    def pf_rec(g, k):
        pltpu.async_copy(rec_hbm.at[base + g], recb[k], isem[k])
        pltpu.async_copy(wt_hbm.at[base + g], wbuf[k], isem[k])

    def wait_rec(k):
        pltpu.make_async_copy(rec_hbm.at[base], recb[k], isem[k]).wait()
        pltpu.make_async_copy(wt_hbm.at[base], wbuf[k], isem[k]).wait()

    NSPLIT = 4
    SROWS = CHUNK // NSPLIT

    def issue_gather(g, k):
        del g
        for p in range(NSPLIT):
            sl = pl.ds(p * SROWS, SROWS)
            pltpu.async_copy(z_hbm.at[recb[k].at[0, sl]],
                             rows[k].at[sl], gsem[k])

    def wait_gather(k):
        for p in range(NSPLIT):
            sl = pl.ds(p * SROWS, SROWS)
            pltpu.make_async_copy(z_hbm.at[recb[k].at[0, sl]],
                                  rows[k].at[sl], gsem[k]).wait()

    def wait_scatter(k):
        pltpu.async_copy(rows[k], acc.at[sidx.at[k]], ssem[k], add=True)

    def process(k):
        def group_body(gg, carry):
            w16 = wbuf[k][0, pl.ds(gg * LANES, LANES)]
            e0 = gg * LANES
            for i in range(LANES):
                wsplat = jnp.full((LANES,), w16[i])
                for dd in range(D // LANES):
                    sl = pl.ds(dd * LANES, LANES)
                    rows[k][e0 + i, sl] = rows[k][e0 + i, sl] * wsplat
            return carry
        lax.fori_loop(0, GROUPS, group_body, 0)
        for j in range(GROUPS):
            sl = pl.ds(j * LANES, LANES)
            sidx[k, sl] = recb[k][1, sl]
        pltpu.async_copy(rows[k], acc.at[sidx.at[k]], ssem[k], add=True)

    def zero_body(i, carry):
        rows0[i // (D // LANES), pl.ds((i % (D // LANES)) * LANES, LANES)] = (
            jnp.zeros((LANES,), jnp.float32))
        return carry
    lax.fori_loop(0, ZROWS * (D // LANES), zero_body, 0)
    for j in range(ROWS_PER_TILE // ZROWS):
        pltpu.sync_copy(rows0, acc.at[pl.ds(s * ROWS_PER_TILE + j * ZROWS, ZROWS)])
    plsc.subcore_barrier()

    pf_rec(0, 0)
    pf_rec(1, 1)
    wait_rec(0)
    issue_gather(0, 0)

    def t_body(t, carry):
        g = t * NBUF
        pf_rec(g + 2, 2)
        wait_rec(1)

        @pl.when(t > 0)
        def _():
            wait_scatter(1)
        issue_gather(g + 1, 1)
        wait_gather(0)
        process(0)
        pf_rec(g + 3, 0)
        wait_rec(2)

        @pl.when(t > 0)
        def _():
            wait_scatter(2)
        issue_gather(g + 2, 2)
        wait_gather(1)
        process(1)
        pf_rec(g + 4, 1)
        wait_rec(0)
        wait_scatter(0)
        issue_gather(g + 3, 0)
        wait_gather(2)
        process(2)
        return carry

    lax.fori_loop(0, MAIN_T, t_body, 0)

    wait_rec(1)
    wait_scatter(1)
    issue_gather(N_CHUNKS - 1, 1)
    wait_gather(0)
    process(0)
    wait_scatter(2)
    wait_gather(1)
    process(1)
    wait_scatter(0)
    wait_scatter(1)

    plsc.subcore_barrier()

    pltpu.sync_copy(acc.at[pl.ds(s * ROWS_PER_TILE, ROWS_PER_TILE)],
                    out_hbm.at[c, pl.ds(s * ROWS_PER_TILE, ROWS_PER_TILE)])


@functools.cache
def _get_sc_kernel():
    mesh = plsc.VectorSubcoreMesh(core_axis_name="c", subcore_axis_name="s",
                                  num_cores=NC, num_subcores=NS)
    return pl.kernel(
        _sc_body,
        out_type=jax.ShapeDtypeStruct((NC, N_PAD, D), jnp.float32),
        mesh=mesh,
        scratch_types=[
            pltpu.VMEM((CHUNK, D), jnp.float32),
            pltpu.VMEM((CHUNK, D), jnp.float32),
            pltpu.VMEM((CHUNK, D), jnp.float32),
            pltpu.VMEM((2, CHUNK), jnp.int32),
            pltpu.VMEM((2, CHUNK), jnp.int32),
            pltpu.VMEM((2, CHUNK), jnp.int32),
            pltpu.VMEM((1, CHUNK), jnp.float32),
            pltpu.VMEM((1, CHUNK), jnp.float32),
            pltpu.VMEM((1, CHUNK), jnp.float32),
            pltpu.VMEM((NBUF, CHUNK), jnp.int32),
            pltpu.VMEM_SHARED((N_PAD, D), jnp.float32),
            pltpu.SemaphoreType.DMA,
            pltpu.SemaphoreType.DMA,
            pltpu.SemaphoreType.DMA,
            pltpu.SemaphoreType.DMA,
            pltpu.SemaphoreType.DMA,
            pltpu.SemaphoreType.DMA,
            pltpu.SemaphoreType.DMA,
            pltpu.SemaphoreType.DMA,
            pltpu.SemaphoreType.DMA,
        ],
    )


def _sc_aggregate(z, rec, wt):
    return _get_sc_kernel()(z, rec, wt)


def _lin_body(x_ref, w_ref, b_ref, o_ref):
    o_ref[...] = lax.dot_general(
        x_ref[...], w_ref[...], (((1,), (1,)), ((), ())),
        preferred_element_type=jnp.float32) + b_ref[...]


def _linear(x, w, b):
    return pl.pallas_call(
        _lin_body,
        out_shape=jax.ShapeDtypeStruct((N_PAD, D), jnp.float32),
    )(x, w, b.reshape(1, D))


def _mid_body(acc_ref, a_ref, g_ref, be_ref, w_ref, b_ref, h_ref, z_ref):
    agg = acc_ref[0, :N_NODES, :] + acc_ref[1, :N_NODES, :]
    a = a_ref[0, 0]
    act = jnp.where(agg >= 0, agg, a * agg)
    mean = jnp.sum(act, axis=0, keepdims=True) * (1.0 / N_NODES)
    cen = act - mean
    var = jnp.sum(cen * cen, axis=0, keepdims=True) * (1.0 / N_NODES)
    h = g_ref[...] * cen * lax.rsqrt(var + EPS) + be_ref[...]
    h_ref[...] = h
    z_ref[pl.ds(0, N_NODES), :] = lax.dot_general(
        h, w_ref[...], (((1,), (1,)), ((), ())),
        preferred_element_type=jnp.float32) + b_ref[...]
    z_ref[pl.ds(N_NODES, N_PAD - N_NODES), :] = jnp.zeros(
        (N_PAD - N_NODES, D), jnp.float32)


def _mid(acc, a, g, be, w1, b1):
    return pl.pallas_call(
        _mid_body,
        out_shape=[jax.ShapeDtypeStruct((N_NODES, D), jnp.float32),
                   jax.ShapeDtypeStruct((N_PAD, D), jnp.float32)],
    )(acc, a.reshape(1, 1), g.reshape(1, D), be.reshape(1, D),
      w1, b1.reshape(1, D))


def _final_body(acc_ref, hin_ref, a_ref, g_ref, be_ref, o_ref):
    agg = acc_ref[0, :N_NODES, :] + acc_ref[1, :N_NODES, :]
    a = a_ref[0, 0]
    act = jnp.where(agg >= 0, agg, a * agg)
    h = act + hin_ref[...]
    mean = jnp.sum(h, axis=0, keepdims=True) * (1.0 / N_NODES)
    cen = h - mean
    var = jnp.sum(cen * cen, axis=0, keepdims=True) * (1.0 / N_NODES)
    hn = g_ref[...] * cen * lax.rsqrt(var + EPS) + be_ref[...]
    o_ref[...] = jnp.where(hn > 0, hn, jnp.exp(hn) - 1.0)


def _final(acc, hin, a, g, be):
    return pl.pallas_call(
        _final_body,
        out_shape=jax.ShapeDtypeStruct((N_NODES, D), jnp.float32),
    )(acc, hin, a.reshape(1, 1), g.reshape(1, D), be.reshape(1, D))


def kernel(seq, adj_edge_index, adj_edge_weight, W0, b0, a0, g0, be0,
           W1, b1, a1, g1, be1):
    ei = adj_edge_index.astype(jnp.int32)
    epad = E_PAD - N_EDGES
    zpad_i = jnp.zeros((epad,), jnp.int32)
    dst = jnp.concatenate([ei[0], zpad_i]).reshape(NW * N_CHUNKS, CHUNK)
    src = jnp.concatenate([ei[1], zpad_i]).reshape(NW * N_CHUNKS, CHUNK)
    w = jnp.concatenate([adj_edge_weight.astype(jnp.float32),
                         jnp.zeros((epad,), jnp.float32)])
    rec = jnp.stack([src, dst], axis=1)
    wt = w.reshape(NW * N_CHUNKS, 1, CHUNK)
    seqp = jnp.pad(seq, ((0, N_PAD - N_NODES), (0, 0)))

    z0 = _linear(seqp, W0, b0)
    acc0 = _sc_aggregate(z0, rec, wt)
    h0, z1 = _mid(acc0, a0, g0, be0, W1, b1)
    acc1 = _sc_aggregate(z1, rec, wt)
    return _final(acc1, h0, a1, g1, be1)


# speedup vs baseline: 1.1652x; 1.1652x over previous
import functools

import jax
import jax.numpy as jnp
from jax import lax
from jax.experimental import pallas as pl
from jax.experimental.pallas import tpu as pltpu
from jax.experimental.pallas import tpu_sc as plsc

N_NODES = 10000
N_EDGES = 320000
D = 128
EPS = 1e-5

NC = 2
NS = 16
NW = NC * NS
CHUNK = 80
N_CHUNKS = 128
E_PAD = NW * N_CHUNKS * CHUNK
N_PAD = 10240
ROWS_PER_TILE = N_PAD // NS
ZROWS = CHUNK
LANES = 16
DSC = D
GROUPS = CHUNK // LANES
NBUF = 3
MAIN_T = N_CHUNKS // NBUF


def _sc_body(z_hbm, rec_hbm, wt_hbm, out_hbm,
             rows0, rows1, rows2, rb0, rb1, rb2,
             rec0, rec1, rec2, w0, w1, w2, sidx, acc,
             g0, g1, g2, s0, s1, s2, i0, i1, i2):
    c = lax.axis_index("c")
    s = lax.axis_index("s")
    wid = s * NC + c
    base = wid * N_CHUNKS

    rows = (rows0, rows1, rows2)
    rbuf = (rb0, rb1, rb2)
    recb = (rec0, rec1, rec2)
    wbuf = (w0, w1, w2)
    gsem = (g0, g1, g2)
    ssem = (s0, s1, s2)
    isem = (i0, i1, i2)

    def pf_rec(g, k):
        pltpu.async_copy(rec_hbm.at[base + g], recb[k], isem[k])
        pltpu.async_copy(wt_hbm.at[base + g], wbuf[k], isem[k])

    def wait_rec(k):
        pltpu.make_async_copy(rec_hbm.at[base], recb[k], isem[k]).wait()
        pltpu.make_async_copy(wt_hbm.at[base], wbuf[k], isem[k]).wait()

    def issue_gather(g, k):
        del g
        pltpu.async_copy(z_hbm.at[recb[k].at[0]], rbuf[k], gsem[k])

    def wait_gather(k):
        pltpu.make_async_copy(z_hbm.at[recb[k].at[0]], rbuf[k], gsem[k]).wait()

    def wait_scatter(k):
        pltpu.make_async_copy(rows[k], acc.at[sidx.at[k]], ssem[k]).wait()

    def process(k):
        def group_body(gg, carry):
            w16 = wbuf[k][0, pl.ds(gg * LANES, LANES)]
            e0 = gg * LANES
            for i in range(LANES):
                wsplat = jnp.full((LANES,), w16[i])
                for dd in range(DSC // (2 * LANES)):
                    m16 = rbuf[k][e0 + i, pl.ds(dd * LANES, LANES)]
                    mb = plsc.bitcast(m16, jnp.bfloat16)
                    va, vb = plsc.unpack(mb, format=plsc.PackFormat.INTERLEAVED)
                    rows[k][e0 + i, pl.ds(dd * 2 * LANES, LANES)] = va * wsplat
                    rows[k][e0 + i, pl.ds(dd * 2 * LANES + LANES, LANES)] = (
                        vb * wsplat)
            return carry
        lax.fori_loop(0, GROUPS, group_body, 0)
        for j in range(GROUPS):
            sl = pl.ds(j * LANES, LANES)
            sidx[k, sl] = recb[k][1, sl]
        pltpu.async_copy(rows[k], acc.at[sidx.at[k]], ssem[k], add=True)

    def zero_body(i, carry):
        rows0[i // (DSC // LANES), pl.ds((i % (DSC // LANES)) * LANES, LANES)] = (
            jnp.zeros((LANES,), jnp.float32))
        return carry
    lax.fori_loop(0, ZROWS * (DSC // LANES), zero_body, 0)
    for j in range(ROWS_PER_TILE // ZROWS):
        pltpu.sync_copy(rows0, acc.at[pl.ds(s * ROWS_PER_TILE + j * ZROWS, ZROWS)])
    plsc.subcore_barrier()

    pf_rec(0, 0)
    pf_rec(1, 1)
    wait_rec(0)
    issue_gather(0, 0)

    def t_body(t, carry):
        g = t * NBUF
        pf_rec(g + 2, 2)
        wait_rec(1)

        @pl.when(t > 0)
        def _():
            wait_scatter(1)
        issue_gather(g + 1, 1)
        wait_gather(0)
        process(0)
        pf_rec(g + 3, 0)
        wait_rec(2)

        @pl.when(t > 0)
        def _():
            wait_scatter(2)
        issue_gather(g + 2, 2)
        wait_gather(1)
        process(1)
        pf_rec(g + 4, 1)
        wait_rec(0)
        wait_scatter(0)
        issue_gather(g + 3, 0)
        wait_gather(2)
        process(2)
        return carry

    lax.fori_loop(0, MAIN_T, t_body, 0)

    wait_rec(1)
    wait_scatter(1)
    issue_gather(N_CHUNKS - 1, 1)
    wait_gather(0)
    process(0)
    wait_scatter(2)
    wait_gather(1)
    process(1)
    wait_scatter(0)
    wait_scatter(1)

    plsc.subcore_barrier()

    pltpu.sync_copy(acc.at[pl.ds(s * ROWS_PER_TILE, ROWS_PER_TILE)],
                    out_hbm.at[c, pl.ds(s * ROWS_PER_TILE, ROWS_PER_TILE)])


@functools.cache
def _get_sc_kernel():
    mesh = plsc.VectorSubcoreMesh(core_axis_name="c", subcore_axis_name="s",
                                  num_cores=NC, num_subcores=NS)
    return pl.kernel(
        _sc_body,
        out_type=jax.ShapeDtypeStruct((NC, N_PAD, DSC), jnp.float32),
        mesh=mesh,
        compiler_params=pltpu.CompilerParams(needs_layout_passes=False,
                                             use_tc_tiling_on_sc=False),
        scratch_types=[
            pltpu.VMEM((CHUNK, DSC), jnp.float32),
            pltpu.VMEM((CHUNK, DSC), jnp.float32),
            pltpu.VMEM((CHUNK, DSC), jnp.float32),
            pltpu.VMEM((CHUNK, DSC // 2), jnp.int32),
            pltpu.VMEM((CHUNK, DSC // 2), jnp.int32),
            pltpu.VMEM((CHUNK, DSC // 2), jnp.int32),
            pltpu.VMEM((2, CHUNK), jnp.int32),
            pltpu.VMEM((2, CHUNK), jnp.int32),
            pltpu.VMEM((2, CHUNK), jnp.int32),
            pltpu.VMEM((1, CHUNK), jnp.float32),
            pltpu.VMEM((1, CHUNK), jnp.float32),
            pltpu.VMEM((1, CHUNK), jnp.float32),
            pltpu.VMEM((NBUF, CHUNK), jnp.int32),
            pltpu.VMEM_SHARED((N_PAD, DSC), jnp.float32),
            pltpu.SemaphoreType.DMA,
            pltpu.SemaphoreType.DMA,
            pltpu.SemaphoreType.DMA,
            pltpu.SemaphoreType.DMA,
            pltpu.SemaphoreType.DMA,
            pltpu.SemaphoreType.DMA,
            pltpu.SemaphoreType.DMA,
            pltpu.SemaphoreType.DMA,
            pltpu.SemaphoreType.DMA,
        ],
    )


def _sc_aggregate(z, rec, wt):
    return _get_sc_kernel()(z, rec, wt)


def _lin_body(x_ref, w_ref, b_ref, o_ref):
    o_ref[...] = lax.dot_general(
        x_ref[...], w_ref[...], (((1,), (1,)), ((), ())),
        preferred_element_type=jnp.float32) + b_ref[...]


def _linear(x, w, b):
    return pl.pallas_call(
        _lin_body,
        out_shape=jax.ShapeDtypeStruct((N_PAD, D), jnp.float32),
    )(x, w, b.reshape(1, D))


def _mid_body(acc_ref, a_ref, g_ref, be_ref, w_ref, b_ref, h_ref, z_ref):
    agg = acc_ref[0, :N_NODES, :] + acc_ref[1, :N_NODES, :]
    a = a_ref[0, 0]
    act = jnp.where(agg >= 0, agg, a * agg)
    mean = jnp.sum(act, axis=0, keepdims=True) * (1.0 / N_NODES)
    cen = act - mean
    var = jnp.sum(cen * cen, axis=0, keepdims=True) * (1.0 / N_NODES)
    h = g_ref[...] * cen * lax.rsqrt(var + EPS) + be_ref[...]
    h_ref[...] = h
    z_ref[pl.ds(0, N_NODES), :] = lax.dot_general(
        h, w_ref[...], (((1,), (1,)), ((), ())),
        preferred_element_type=jnp.float32) + b_ref[...]
    z_ref[pl.ds(N_NODES, N_PAD - N_NODES), :] = jnp.zeros(
        (N_PAD - N_NODES, D), jnp.float32)


def _mid(acc, a, g, be, w1, b1):
    return pl.pallas_call(
        _mid_body,
        out_shape=[jax.ShapeDtypeStruct((N_NODES, D), jnp.float32),
                   jax.ShapeDtypeStruct((N_PAD, D), jnp.float32)],
    )(acc, a.reshape(1, 1), g.reshape(1, D), be.reshape(1, D),
      w1, b1.reshape(1, D))


def _final_body(acc_ref, hin_ref, a_ref, g_ref, be_ref, o_ref):
    agg = acc_ref[0, :N_NODES, :] + acc_ref[1, :N_NODES, :]
    a = a_ref[0, 0]
    act = jnp.where(agg >= 0, agg, a * agg)
    h = act + hin_ref[...]
    mean = jnp.sum(h, axis=0, keepdims=True) * (1.0 / N_NODES)
    cen = h - mean
    var = jnp.sum(cen * cen, axis=0, keepdims=True) * (1.0 / N_NODES)
    hn = g_ref[...] * cen * lax.rsqrt(var + EPS) + be_ref[...]
    o_ref[...] = jnp.where(hn > 0, hn, jnp.exp(hn) - 1.0)


def _final(acc, hin, a, g, be):
    return pl.pallas_call(
        _final_body,
        out_shape=jax.ShapeDtypeStruct((N_NODES, D), jnp.float32),
    )(acc, hin, a.reshape(1, 1), g.reshape(1, D), be.reshape(1, D))


_PERM = sum(([32 * b + j, 32 * b + 16 + j] for b in range(D // 32)
             for j in range(16)), [])


def _to_bf16_interleaved(z):
    zb = z[:, jnp.array(_PERM, dtype=jnp.int32)].astype(jnp.bfloat16)
    return lax.bitcast_convert_type(zb.reshape(N_PAD, D // 2, 2), jnp.int32)
def kernel(seq, adj_edge_index, adj_edge_weight, W0, b0, a0, g0, be0,
           W1, b1, a1, g1, be1):
    ei = adj_edge_index.astype(jnp.int32)
    epad = E_PAD - N_EDGES
    zpad_i = jnp.zeros((epad,), jnp.int32)
    dst = jnp.concatenate([ei[0], zpad_i]).reshape(NW * N_CHUNKS, CHUNK)
    src = jnp.concatenate([ei[1], zpad_i]).reshape(NW * N_CHUNKS, CHUNK)
    w = jnp.concatenate([adj_edge_weight.astype(jnp.float32),
                         jnp.zeros((epad,), jnp.float32)])
    rec = jnp.stack([src, dst], axis=1)
    wt = w.reshape(NW * N_CHUNKS, 1, CHUNK)
    seqp = jnp.pad(seq, ((0, N_PAD - N_NODES), (0, 0)))

    z0 = _linear(seqp, W0, b0)
    acc0 = _sc_aggregate(_to_bf16_interleaved(z0), rec, wt)
    h0, z1 = _mid(acc0, a0, g0, be0, W1, b1)
    acc1 = _sc_aggregate(_to_bf16_interleaved(z1), rec, wt)
    return _final(acc1, h0, a1, g1, be1)
